# baseline (speedup 1.0000x reference)
import functools

import jax
import jax.numpy as jnp
from jax import lax
from jax.experimental import pallas as pl
from jax.experimental.pallas import tpu as pltpu
from jax.experimental.pallas import tpu_sc as plsc

_CHUNK = 128
_NW = 32
_BATCH = 8
_BN = 2048


def _worker_id(cid, sid):
    return sid * 2 + cid


def _sc_degree(dst2p, vals2, n2):
    nchunks = dst2p.shape[0]
    per_w = nchunks // _NW
    nbat = per_w // _BATCH
    mesh = plsc.VectorSubcoreMesh(core_axis_name="c", subcore_axis_name="s")

    zc = 512

    @functools.partial(
        pl.kernel,
        out_type=jax.ShapeDtypeStruct((2 * n2,), jnp.float32),
        mesh=mesh,
        scratch_types=[
            pltpu.VMEM((_BATCH, _CHUNK), jnp.int32),
            pltpu.VMEM((_BATCH, _CHUNK), jnp.float32),
            pltpu.VMEM((zc,), jnp.float32),
            pltpu.VMEM_SHARED((n2,), jnp.float32),
            pltpu.SemaphoreType.DMA,
        ],
    )
    def k(dst_hbm, val_hbm, out_hbm, db, vb, buf, acc, sem):
        cid = lax.axis_index("c")
        sid = lax.axis_index("s")
        wid = _worker_id(cid, sid)

        def fill_zero(i, _):
            buf[pl.ds(i * 16, 16)] = jnp.zeros((16,), jnp.float32)
            return 0

        lax.fori_loop(0, zc // 16, fill_zero, 0)

        nzc = n2 // zc

        def zero_step(t, _):
            off = (sid + 16 * t) * zc
            pltpu.sync_copy(buf, acc.at[pl.ds(off, zc)])
            return 0

        lax.fori_loop(0, (nzc - sid + 15) // 16, zero_step, 0)
        plsc.subcore_barrier()

        base = wid * per_w

        def batch(t, _):
            cbase = base + t * _BATCH
            pltpu.sync_copy(dst_hbm.at[pl.ds(cbase, _BATCH)], db)
            pltpu.sync_copy(val_hbm.at[pl.ds(cbase, _BATCH)], vb)
            for j in range(_BATCH):
                pltpu.sync_copy(vb.at[j], acc.at[db.at[j]], add=True)
            return 0

        lax.fori_loop(0, nbat, batch, 0)
        plsc.subcore_barrier()

        def out_step(t, _):
            off = (sid + 16 * t) * zc
            pltpu.sync_copy(acc.at[pl.ds(off, zc)],
                            out_hbm.at[pl.ds(cid * n2 + off, zc)])
            return 0

        lax.fori_loop(0, (nzc - sid + 15) // 16, out_step, 0)

    return k(dst2p, vals2)


_SUB = 64
_T = 16


def _sc_agg(hs, src2, dst2, n2):
    nh, w = hs.shape
    nsub = src2.shape[0]
    per_w = nsub // _NW
    nbat = per_w // _T
    mesh = plsc.VectorSubcoreMesh(core_axis_name="c", subcore_axis_name="s")

    zr = 64

    @functools.partial(
        pl.kernel,
        out_type=jax.ShapeDtypeStruct((2, n2, w), jnp.float32),
        mesh=mesh,
        scratch_types=[
            pltpu.VMEM((_T, _SUB), jnp.int32),
            pltpu.VMEM((_T, _SUB), jnp.int32),
            pltpu.VMEM((4, _SUB, w), jnp.float32),
            pltpu.VMEM((zr, w), jnp.float32),
            pltpu.VMEM_SHARED((n2, w), jnp.float32),
            pltpu.SemaphoreType.DMA,
            pltpu.SemaphoreType.DMA,
            pltpu.SemaphoreType.DMA,
            pltpu.SemaphoreType.DMA,
            pltpu.SemaphoreType.DMA,
            pltpu.SemaphoreType.DMA,
            pltpu.SemaphoreType.DMA,
            pltpu.SemaphoreType.DMA,
        ],
    )
    def k(h_hbm, src_hbm, dst_hbm, out_hbm, sb, db, rows, buf, acc,
          g0, g1, g2, g3, s0, s1, s2, s3):
        cid = lax.axis_index("c")
        sid = lax.axis_index("s")
        wid = _worker_id(cid, sid)
        gsem = (g0, g1, g2, g3)
        ssem = (s0, s1, s2, s3)

        def zrow(r, _):
            def zcol(c_, __):
                buf[r, pl.ds(c_ * 16, 16)] = jnp.zeros((16,), jnp.float32)
                return 0
            lax.fori_loop(0, w // 16, zcol, 0)
            return 0

        lax.fori_loop(0, zr, zrow, 0)

        nzc = n2 // zr

        def zero_step(t, _):
            off = (sid + 16 * t) * zr
            pltpu.sync_copy(buf, acc.at[pl.ds(off, zr)])
            return 0

        lax.fori_loop(0, (nzc - sid + 15) // 16, zero_step, 0)
        plsc.subcore_barrier()

        base = wid * per_w

        def batch(t, _):
            cbase = base + t * _T
            pltpu.sync_copy(src_hbm.at[pl.ds(cbase, _T)], sb)
            pltpu.sync_copy(dst_hbm.at[pl.ds(cbase, _T)], db)
            g = [None] * 4
            s = [None] * 4
            for j in range(2):
                g[j] = pltpu.async_copy(
                    h_hbm.at[sb.at[j]], rows.at[j], gsem[j])
            for j in range(_T):
                b = j & 3
                g[b].wait()
                s[b] = pltpu.async_copy(
                    rows.at[b], acc.at[db.at[j]], ssem[b], add=True)
                jn = j + 2
                if jn < _T:
                    bn = jn & 3
                    if s[bn] is not None:
                        s[bn].wait()
                    g[bn] = pltpu.async_copy(
                        h_hbm.at[sb.at[jn]], rows.at[bn], gsem[bn])
            for j in range(_T - 4, _T):
                s[j & 3].wait()
            return 0

        lax.fori_loop(0, nbat, batch, 0)
        plsc.subcore_barrier()

        def out_step(t, _):
            off = (sid + 16 * t) * zr
            pltpu.sync_copy(acc.at[pl.ds(off, zr)],
                            out_hbm.at[cid, pl.ds(off, zr)])
            return 0

        lax.fori_loop(0, (nzc - sid + 15) // 16, out_step, 0)

    return k(hs, src2, dst2)


def _tc_prep(p0, p1, x):
    n2, d = x.shape

    def body(p0_ref, p1_ref, x_ref, dinv_ref, xs_ref):
        deg = p0_ref[...] + p1_ref[...] + 1.0
        dv = lax.rsqrt(deg)
        dinv_ref[...] = dv
        xs_ref[...] = x_ref[...] * dv

    return pl.pallas_call(
        body,
        grid=(n2 // _BN,),
        in_specs=[
            pl.BlockSpec((_BN, 1), lambda i: (i, 0)),
            pl.BlockSpec((_BN, 1), lambda i: (i, 0)),
            pl.BlockSpec((_BN, d), lambda i: (i, 0)),
        ],
        out_specs=[
            pl.BlockSpec((_BN, 1), lambda i: (i, 0)),
            pl.BlockSpec((_BN, d), lambda i: (i, 0)),
        ],
        out_shape=[
            jax.ShapeDtypeStruct((n2, 1), jnp.float32),
            jax.ShapeDtypeStruct((n2, d), jnp.float32),
        ],
    )(p0, p1, x)


def _tc_layer(parts, ys, dinv, W, b, n_real):
    n2, d = ys.shape
    h = W.shape[1]

    def body(q0_ref, q1_ref, ys_ref, dv_ref, w_ref, b_ref, out_ref):
        t = q0_ref[0] + q1_ref[0] + ys_ref[...]
        t = jnp.dot(t, w_ref[...], preferred_element_type=jnp.float32)
        dv = dv_ref[...]
        out = jnp.maximum(dv * t + b_ref[...], 0.0)
        row = (pl.program_id(0) * _BN
               + lax.broadcasted_iota(jnp.int32, (_BN, 1), 0))
        out_ref[...] = jnp.where(row < n_real, dv * out, 0.0)

    return pl.pallas_call(
        body,
        grid=(n2 // _BN,),
        in_specs=[
            pl.BlockSpec((1, _BN, d), lambda i: (0, i, 0)),
            pl.BlockSpec((1, _BN, d), lambda i: (1, i, 0)),
            pl.BlockSpec((_BN, d), lambda i: (i, 0)),
            pl.BlockSpec((_BN, 1), lambda i: (i, 0)),
            pl.BlockSpec((d, h), lambda i: (0, 0)),
            pl.BlockSpec((1, h), lambda i: (0, 0)),
        ],
        out_specs=pl.BlockSpec((_BN, h), lambda i: (i, 0)),
        out_shape=jax.ShapeDtypeStruct((n2, h), jnp.float32),
    )(parts, parts, ys, dinv, W, b)


def _tc_final(parts, ys3, dinv, W3, b, n_real):
    n2, d = ys3.shape
    d3 = W3.shape[1]

    def body(r0_ref, r1_ref, ys_ref, dv_ref, w3_ref, b_ref, h_ref, xg_ref):
        t = r0_ref[0] + r1_ref[0] + ys_ref[...]
        t = jnp.dot(t, w3_ref[...], preferred_element_type=jnp.float32)
        hb = dv_ref[...] * t + b_ref[...]
        h_ref[...] = hb

        @pl.when(pl.program_id(0) == 0)
        def _():
            xg_ref[...] = jnp.zeros_like(xg_ref)

        row = (pl.program_id(0) * _BN
               + lax.broadcasted_iota(jnp.int32, (_BN, 1), 0))
        hm = jnp.where(row < n_real, hb, 0.0)
        xg_ref[...] += jnp.sum(hm, axis=0, keepdims=True) * (1.0 / n_real)

    return pl.pallas_call(
        body,
        grid=(n2 // _BN,),
        in_specs=[
            pl.BlockSpec((1, _BN, d), lambda i: (0, i, 0)),
            pl.BlockSpec((1, _BN, d), lambda i: (1, i, 0)),
            pl.BlockSpec((_BN, d), lambda i: (i, 0)),
            pl.BlockSpec((_BN, 1), lambda i: (i, 0)),
            pl.BlockSpec((d, d3), lambda i: (0, 0)),
            pl.BlockSpec((1, d3), lambda i: (0, 0)),
        ],
        out_specs=[
            pl.BlockSpec((_BN, d3), lambda i: (i, 0)),
            pl.BlockSpec((1, d3), lambda i: (0, 0)),
        ],
        out_shape=[
            jax.ShapeDtypeStruct((n2, d3), jnp.float32),
            jax.ShapeDtypeStruct((1, d3), jnp.float32),
        ],
    )(parts, parts, ys3, dinv, W3, b)


def kernel(x, edge_index, W1, b1, W2, b2, W3, b3):
    n, d_in = x.shape
    e = edge_index.shape[1]
    nchunks = e // _CHUNK

    n2 = -(-n // _BN) * _BN
    x_p = jnp.concatenate(
        [x, jnp.zeros((n2 - n, d_in), x.dtype)], axis=0)

    npc = -(-nchunks // (_NW * _BATCH)) * (_NW * _BATCH)
    pad = npc * _CHUNK - e
    pad_ar = jnp.arange(pad, dtype=edge_index.dtype)
    src_p = jnp.concatenate([edge_index[0], n + pad_ar % (n2 - n)])
    dst_p = jnp.concatenate([edge_index[1], pad_ar % n2])
    src2 = src_p.reshape(npc * (_CHUNK // _SUB), _SUB)
    dst2s = dst_p.reshape(npc * (_CHUNK // _SUB), _SUB)
    dst2p = dst_p.reshape(npc, _CHUNK)
    vals2 = jnp.concatenate(
        [jnp.ones((e,), jnp.float32),
         jnp.zeros((pad,), jnp.float32)]).reshape(npc, _CHUNK)

    deg_parts = _sc_degree(dst2p, vals2, n2).reshape(2, n2)
    p0 = deg_parts[0].reshape(n2, 1)
    p1 = deg_parts[1].reshape(n2, 1)
    dinv, xs = _tc_prep(p0, p1, x_p)

    s1 = _sc_agg(xs, src2, dst2s, n2)
    ys2 = _tc_layer(s1, xs, dinv, W1, b1.reshape(1, -1), n)

    s2 = _sc_agg(ys2, src2, dst2s, n2)
    ys3 = _tc_layer(s2, ys2, dinv, W2, b2.reshape(1, -1), n)

    s3 = _sc_agg(ys3, src2, dst2s, n2)
    h, xg = _tc_final(s3, ys3, dinv, W3, b3.reshape(1, -1), n)
    return (h[:n], xg)


# speedup vs baseline: 1.1456x; 1.1456x over previous
import functools

import jax
import jax.numpy as jnp
from jax import lax
from jax.experimental import pallas as pl
from jax.experimental.pallas import tpu as pltpu
from jax.experimental.pallas import tpu_sc as plsc

_CHUNK = 128
_NW = 32
_BATCH = 16
_BN = 2048


def _worker_id(cid, sid):
    return sid * 2 + cid


def _sc_degree(dst2p, vals2, n2):
    nchunks = dst2p.shape[0]
    per_w = nchunks // _NW
    nbat = per_w // _BATCH
    mesh = plsc.VectorSubcoreMesh(core_axis_name="c", subcore_axis_name="s")

    zc = 512

    @functools.partial(
        pl.kernel,
        out_type=jax.ShapeDtypeStruct((2 * n2,), jnp.float32),
        mesh=mesh,
        scratch_types=[
            pltpu.VMEM((_BATCH, _CHUNK), jnp.int32),
            pltpu.VMEM((_BATCH, _CHUNK), jnp.float32),
            pltpu.VMEM((zc,), jnp.float32),
            pltpu.VMEM_SHARED((n2,), jnp.float32),
            pltpu.SemaphoreType.DMA,
        ],
    )
    def k(dst_hbm, val_hbm, out_hbm, db, vb, buf, acc, sem):
        cid = lax.axis_index("c")
        sid = lax.axis_index("s")
        wid = _worker_id(cid, sid)

        def fill_zero(i, _):
            buf[pl.ds(i * 16, 16)] = jnp.zeros((16,), jnp.float32)
            return 0

        lax.fori_loop(0, zc // 16, fill_zero, 0)

        nzc = n2 // zc

        def zero_step(t, _):
            off = (sid + 16 * t) * zc
            pltpu.sync_copy(buf, acc.at[pl.ds(off, zc)])
            return 0

        lax.fori_loop(0, (nzc - sid + 15) // 16, zero_step, 0)
        plsc.subcore_barrier()

        base = wid * per_w

        def batch(t, _):
            cbase = base + t * _BATCH
            pltpu.sync_copy(dst_hbm.at[pl.ds(cbase, _BATCH)], db)
            pltpu.sync_copy(val_hbm.at[pl.ds(cbase, _BATCH)], vb)
            for j in range(_BATCH):
                pltpu.sync_copy(vb.at[j], acc.at[db.at[j]], add=True)
            return 0

        lax.fori_loop(0, nbat, batch, 0)
        plsc.subcore_barrier()

        def out_step(t, _):
            off = (sid + 16 * t) * zc
            pltpu.sync_copy(acc.at[pl.ds(off, zc)],
                            out_hbm.at[pl.ds(cid * n2 + off, zc)])
            return 0

        lax.fori_loop(0, (nzc - sid + 15) // 16, out_step, 0)

    return k(dst2p, vals2)


def _sc_agg(hs, src2, dst2, n2):
    nh, w = hs.shape
    nchunks = src2.shape[0]
    per_w = nchunks // _NW
    nbat = per_w // _BATCH
    mesh = plsc.VectorSubcoreMesh(core_axis_name="c", subcore_axis_name="s")

    zr = 64

    @functools.partial(
        pl.kernel,
        out_type=jax.ShapeDtypeStruct((2, n2, w), jnp.float32),
        mesh=mesh,
        scratch_types=[
            pltpu.VMEM((_BATCH, _CHUNK), jnp.int32),
            pltpu.VMEM((_BATCH, _CHUNK), jnp.int32),
            pltpu.VMEM((2, _CHUNK, w), jnp.float32),
            pltpu.VMEM((zr, w), jnp.float32),
            pltpu.VMEM_SHARED((n2, w), jnp.float32),
            pltpu.SemaphoreType.DMA,
            pltpu.SemaphoreType.DMA,
        ],
    )
    def k(h_hbm, src_hbm, dst_hbm, out_hbm, sb, db, rows, buf, acc,
          sem0, sem1):
        cid = lax.axis_index("c")
        sid = lax.axis_index("s")
        wid = _worker_id(cid, sid)
        sems = (sem0, sem1)

        def zrow(r, _):
            def zcol(c_, __):
                buf[r, pl.ds(c_ * 16, 16)] = jnp.zeros((16,), jnp.float32)
                return 0
            lax.fori_loop(0, w // 16, zcol, 0)
            return 0

        lax.fori_loop(0, zr, zrow, 0)

        nzc = n2 // zr

        def zero_step(t, _):
            off = (sid + 16 * t) * zr
            pltpu.sync_copy(buf, acc.at[pl.ds(off, zr)])
            return 0

        lax.fori_loop(0, (nzc - sid + 15) // 16, zero_step, 0)
        plsc.subcore_barrier()

        base = wid * per_w

        def batch(t, _):
            cbase = base + t * _BATCH
            pltpu.sync_copy(src_hbm.at[pl.ds(cbase, _BATCH)], sb)
            pltpu.sync_copy(dst_hbm.at[pl.ds(cbase, _BATCH)], db)
            descs = [None, None]
            for j in range(2):
                descs[j] = pltpu.async_copy(
                    h_hbm.at[sb.at[j]], rows.at[j], sems[j])
            for j in range(_BATCH):
                b = j & 1
                descs[b].wait()
                pltpu.sync_copy(rows.at[b], acc.at[db.at[j]], add=True)
                if j + 2 < _BATCH:
                    descs[b] = pltpu.async_copy(
                        h_hbm.at[sb.at[j + 2]], rows.at[b], sems[b])
            return 0

        lax.fori_loop(0, nbat, batch, 0)
        plsc.subcore_barrier()

        def out_step(t, _):
            off = (sid + 16 * t) * zr
            pltpu.sync_copy(acc.at[pl.ds(off, zr)],
                            out_hbm.at[cid, pl.ds(off, zr)])
            return 0

        lax.fori_loop(0, (nzc - sid + 15) // 16, out_step, 0)

    return k(hs, src2, dst2)


def _tc_prep(p0, p1, x):
    n2, d = x.shape

    def body(p0_ref, p1_ref, x_ref, dinv_ref, xs_ref):
        deg = p0_ref[...] + p1_ref[...] + 1.0
        dv = lax.rsqrt(deg)
        dinv_ref[...] = dv
        xs_ref[...] = x_ref[...] * dv

    return pl.pallas_call(
        body,
        grid=(n2 // _BN,),
        in_specs=[
            pl.BlockSpec((_BN, 1), lambda i: (i, 0)),
            pl.BlockSpec((_BN, 1), lambda i: (i, 0)),
            pl.BlockSpec((_BN, d), lambda i: (i, 0)),
        ],
        out_specs=[
            pl.BlockSpec((_BN, 1), lambda i: (i, 0)),
            pl.BlockSpec((_BN, d), lambda i: (i, 0)),
        ],
        out_shape=[
            jax.ShapeDtypeStruct((n2, 1), jnp.float32),
            jax.ShapeDtypeStruct((n2, d), jnp.float32),
        ],
    )(p0, p1, x)


def _tc_layer(parts, ys, dinv, W, b, n_real):
    n2, d = ys.shape
    h = W.shape[1]

    def body(q0_ref, q1_ref, ys_ref, dv_ref, w_ref, b_ref, out_ref):
        t = q0_ref[0] + q1_ref[0] + ys_ref[...]
        t = jnp.dot(t, w_ref[...], preferred_element_type=jnp.float32)
        dv = dv_ref[...]
        out = jnp.maximum(dv * t + b_ref[...], 0.0)
        row = (pl.program_id(0) * _BN
               + lax.broadcasted_iota(jnp.int32, (_BN, 1), 0))
        out_ref[...] = jnp.where(row < n_real, dv * out, 0.0)

    return pl.pallas_call(
        body,
        grid=(n2 // _BN,),
        in_specs=[
            pl.BlockSpec((1, _BN, d), lambda i: (0, i, 0)),
            pl.BlockSpec((1, _BN, d), lambda i: (1, i, 0)),
            pl.BlockSpec((_BN, d), lambda i: (i, 0)),
            pl.BlockSpec((_BN, 1), lambda i: (i, 0)),
            pl.BlockSpec((d, h), lambda i: (0, 0)),
            pl.BlockSpec((1, h), lambda i: (0, 0)),
        ],
        out_specs=pl.BlockSpec((_BN, h), lambda i: (i, 0)),
        out_shape=jax.ShapeDtypeStruct((n2, h), jnp.float32),
    )(parts, parts, ys, dinv, W, b)


def _tc_final(parts, ys3, dinv, W3, b, n_real):
    n2, d = ys3.shape
    d3 = W3.shape[1]

    def body(r0_ref, r1_ref, ys_ref, dv_ref, w3_ref, b_ref, h_ref, xg_ref):
        t = r0_ref[0] + r1_ref[0] + ys_ref[...]
        t = jnp.dot(t, w3_ref[...], preferred_element_type=jnp.float32)
        hb = dv_ref[...] * t + b_ref[...]
        h_ref[...] = hb

        @pl.when(pl.program_id(0) == 0)
        def _():
            xg_ref[...] = jnp.zeros_like(xg_ref)

        row = (pl.program_id(0) * _BN
               + lax.broadcasted_iota(jnp.int32, (_BN, 1), 0))
        hm = jnp.where(row < n_real, hb, 0.0)
        xg_ref[...] += jnp.sum(hm, axis=0, keepdims=True) * (1.0 / n_real)

    return pl.pallas_call(
        body,
        grid=(n2 // _BN,),
        in_specs=[
            pl.BlockSpec((1, _BN, d), lambda i: (0, i, 0)),
            pl.BlockSpec((1, _BN, d), lambda i: (1, i, 0)),
            pl.BlockSpec((_BN, d), lambda i: (i, 0)),
            pl.BlockSpec((_BN, 1), lambda i: (i, 0)),
            pl.BlockSpec((d, d3), lambda i: (0, 0)),
            pl.BlockSpec((1, d3), lambda i: (0, 0)),
        ],
        out_specs=[
            pl.BlockSpec((_BN, d3), lambda i: (i, 0)),
            pl.BlockSpec((1, d3), lambda i: (0, 0)),
        ],
        out_shape=[
            jax.ShapeDtypeStruct((n2, d3), jnp.float32),
            jax.ShapeDtypeStruct((1, d3), jnp.float32),
        ],
    )(parts, parts, ys3, dinv, W3, b)


def kernel(x, edge_index, W1, b1, W2, b2, W3, b3):
    n, d_in = x.shape
    e = edge_index.shape[1]
    nchunks = e // _CHUNK

    n2 = -(-n // _BN) * _BN
    x_p = jnp.concatenate(
        [x, jnp.zeros((n2 - n, d_in), x.dtype)], axis=0)

    npc = -(-nchunks // (_NW * _BATCH)) * (_NW * _BATCH)
    pad = npc * _CHUNK - e
    pad_ar = jnp.arange(pad, dtype=edge_index.dtype)
    src_p = jnp.concatenate([edge_index[0], n + pad_ar % (n2 - n)])
    dst_p = jnp.concatenate([edge_index[1], pad_ar % n2])
    src2 = src_p.reshape(npc, _CHUNK)
    dst2p = dst_p.reshape(npc, _CHUNK)
    vals2 = jnp.concatenate(
        [jnp.ones((e,), jnp.float32),
         jnp.zeros((pad,), jnp.float32)]).reshape(npc, _CHUNK)

    deg_parts = _sc_degree(dst2p, vals2, n2).reshape(2, n2)
    p0 = deg_parts[0].reshape(n2, 1)
    p1 = deg_parts[1].reshape(n2, 1)
    dinv, xs = _tc_prep(p0, p1, x_p)

    s1 = _sc_agg(xs, src2, dst2p, n2)
    ys2 = _tc_layer(s1, xs, dinv, W1, b1.reshape(1, -1), n)

    s2 = _sc_agg(ys2, src2, dst2p, n2)
    ys3 = _tc_layer(s2, ys2, dinv, W2, b2.reshape(1, -1), n)

    s3 = _sc_agg(ys3, src2, dst2p, n2)
    h, xg = _tc_final(s3, ys3, dinv, W3, b3.reshape(1, -1), n)
    return (h[:n], xg)


# speedup vs baseline: 1.1531x; 1.0065x over previous
import functools

import jax
import jax.numpy as jnp
from jax import lax
from jax.experimental import pallas as pl
from jax.experimental.pallas import tpu as pltpu
from jax.experimental.pallas import tpu_sc as plsc

_CHUNK = 128
_NW = 32
_BATCH = 16
_BN = 2048


def _worker_id(cid, sid):
    return sid * 2 + cid


def _sc_degree(dst2p, vals2, n2):
    nchunks = dst2p.shape[0]
    per_w = nchunks // _NW
    nbat = per_w // _BATCH
    mesh = plsc.VectorSubcoreMesh(core_axis_name="c", subcore_axis_name="s")

    zc = 512

    @functools.partial(
        pl.kernel,
        out_type=jax.ShapeDtypeStruct((2 * n2,), jnp.float32),
        mesh=mesh,
        scratch_types=[
            pltpu.VMEM((_BATCH, _CHUNK), jnp.int32),
            pltpu.VMEM((_BATCH, _CHUNK), jnp.float32),
            pltpu.VMEM((zc,), jnp.float32),
            pltpu.VMEM_SHARED((n2,), jnp.float32),
            pltpu.SemaphoreType.DMA,
        ],
    )
    def k(dst_hbm, val_hbm, out_hbm, db, vb, buf, acc, sem):
        cid = lax.axis_index("c")
        sid = lax.axis_index("s")
        wid = _worker_id(cid, sid)

        def fill_zero(i, _):
            buf[pl.ds(i * 16, 16)] = jnp.zeros((16,), jnp.float32)
            return 0

        lax.fori_loop(0, zc // 16, fill_zero, 0)

        nzc = n2 // zc

        def zero_step(t, _):
            off = (sid + 16 * t) * zc
            pltpu.sync_copy(buf, acc.at[pl.ds(off, zc)])
            return 0

        lax.fori_loop(0, (nzc - sid + 15) // 16, zero_step, 0)
        plsc.subcore_barrier()

        base = wid * per_w

        def batch(t, _):
            cbase = base + t * _BATCH
            pltpu.sync_copy(dst_hbm.at[pl.ds(cbase, _BATCH)], db)
            pltpu.sync_copy(val_hbm.at[pl.ds(cbase, _BATCH)], vb)
            for j in range(_BATCH):
                pltpu.sync_copy(vb.at[j], acc.at[db.at[j]], add=True)
            return 0

        lax.fori_loop(0, nbat, batch, 0)
        plsc.subcore_barrier()

        def out_step(t, _):
            off = (sid + 16 * t) * zc
            pltpu.sync_copy(acc.at[pl.ds(off, zc)],
                            out_hbm.at[pl.ds(cid * n2 + off, zc)])
            return 0

        lax.fori_loop(0, (nzc - sid + 15) // 16, out_step, 0)

    return k(dst2p, vals2)


def _sc_agg(hs, src2, dst2, n2):
    nh, w = hs.shape
    nchunks = src2.shape[0]
    per_w = nchunks // _NW
    nbat = per_w // _BATCH
    mesh = plsc.VectorSubcoreMesh(core_axis_name="c", subcore_axis_name="s")

    zr = 64

    @functools.partial(
        pl.kernel,
        out_type=jax.ShapeDtypeStruct((2, n2, w), jnp.float32),
        mesh=mesh,
        scratch_types=[
            pltpu.VMEM((_BATCH, _CHUNK), jnp.int32),
            pltpu.VMEM((_BATCH, _CHUNK), jnp.int32),
            pltpu.VMEM((2, _CHUNK, w), jnp.float32),
            pltpu.VMEM((zr, w), jnp.float32),
            pltpu.VMEM_SHARED((n2, w), jnp.float32),
            pltpu.SemaphoreType.DMA,
            pltpu.SemaphoreType.DMA,
        ],
    )
    def k(h_hbm, src_hbm, dst_hbm, out_hbm, sb, db, rows, buf, acc,
          sem0, sem1):
        cid = lax.axis_index("c")
        sid = lax.axis_index("s")
        wid = _worker_id(cid, sid)
        sems = (sem0, sem1)

        def zrow(r, _):
            def zcol(c_, __):
                buf[r, pl.ds(c_ * 16, 16)] = jnp.zeros((16,), jnp.float32)
                return 0
            lax.fori_loop(0, w // 16, zcol, 0)
            return 0

        lax.fori_loop(0, zr, zrow, 0)

        nzc = n2 // zr
        nz = (nzc - sid + 15) // 16

        def zero_step(t, _):
            off = (sid + 16 * t) * zr
            pltpu.async_copy(buf, acc.at[pl.ds(off, zr)], sem1)
            return 0

        lax.fori_loop(0, nz, zero_step, 0)

        def zero_drain(t, _):
            pltpu.make_async_copy(buf, acc.at[pl.ds(sid * zr, zr)],
                                  sem1).wait()
            return 0

        lax.fori_loop(0, nz, zero_drain, 0)
        plsc.subcore_barrier()

        base = wid * per_w

        def batch(t, _):
            cbase = base + t * _BATCH
            pltpu.sync_copy(src_hbm.at[pl.ds(cbase, _BATCH)], sb)
            pltpu.sync_copy(dst_hbm.at[pl.ds(cbase, _BATCH)], db)
            descs = [None, None]
            for j in range(2):
                descs[j] = pltpu.async_copy(
                    h_hbm.at[sb.at[j]], rows.at[j], sems[j])
            for j in range(_BATCH):
                b = j & 1
                descs[b].wait()
                pltpu.sync_copy(rows.at[b], acc.at[db.at[j]], add=True)
                if j + 2 < _BATCH:
                    descs[b] = pltpu.async_copy(
                        h_hbm.at[sb.at[j + 2]], rows.at[b], sems[b])
            return 0

        lax.fori_loop(0, nbat, batch, 0)
        plsc.subcore_barrier()

        def out_step(t, _):
            off = (sid + 16 * t) * zr
            pltpu.async_copy(acc.at[pl.ds(off, zr)],
                             out_hbm.at[cid, pl.ds(off, zr)], sem1)
            return 0

        lax.fori_loop(0, nz, out_step, 0)

        def out_drain(t, _):
            pltpu.make_async_copy(acc.at[pl.ds(sid * zr, zr)],
                                  out_hbm.at[cid, pl.ds(sid * zr, zr)],
                                  sem1).wait()
            return 0

        lax.fori_loop(0, nz, out_drain, 0)

    return k(hs, src2, dst2)


def _tc_prep(p0, p1, x):
    n2, d = x.shape

    def body(p0_ref, p1_ref, x_ref, dinv_ref, xs_ref):
        deg = p0_ref[...] + p1_ref[...] + 1.0
        dv = lax.rsqrt(deg)
        dinv_ref[...] = dv
        xs_ref[...] = x_ref[...] * dv

    return pl.pallas_call(
        body,
        grid=(n2 // _BN,),
        in_specs=[
            pl.BlockSpec((_BN, 1), lambda i: (i, 0)),
            pl.BlockSpec((_BN, 1), lambda i: (i, 0)),
            pl.BlockSpec((_BN, d), lambda i: (i, 0)),
        ],
        out_specs=[
            pl.BlockSpec((_BN, 1), lambda i: (i, 0)),
            pl.BlockSpec((_BN, d), lambda i: (i, 0)),
        ],
        out_shape=[
            jax.ShapeDtypeStruct((n2, 1), jnp.float32),
            jax.ShapeDtypeStruct((n2, d), jnp.float32),
        ],
    )(p0, p1, x)


def _tc_layer(parts, ys, dinv, W, b, n_real):
    n2, d = ys.shape
    h = W.shape[1]

    def body(q0_ref, q1_ref, ys_ref, dv_ref, w_ref, b_ref, out_ref):
        t = q0_ref[0] + q1_ref[0] + ys_ref[...]
        t = jnp.dot(t, w_ref[...], preferred_element_type=jnp.float32)
        dv = dv_ref[...]
        out = jnp.maximum(dv * t + b_ref[...], 0.0)
        row = (pl.program_id(0) * _BN
               + lax.broadcasted_iota(jnp.int32, (_BN, 1), 0))
        out_ref[...] = jnp.where(row < n_real, dv * out, 0.0)

    return pl.pallas_call(
        body,
        grid=(n2 // _BN,),
        in_specs=[
            pl.BlockSpec((1, _BN, d), lambda i: (0, i, 0)),
            pl.BlockSpec((1, _BN, d), lambda i: (1, i, 0)),
            pl.BlockSpec((_BN, d), lambda i: (i, 0)),
            pl.BlockSpec((_BN, 1), lambda i: (i, 0)),
            pl.BlockSpec((d, h), lambda i: (0, 0)),
            pl.BlockSpec((1, h), lambda i: (0, 0)),
        ],
        out_specs=pl.BlockSpec((_BN, h), lambda i: (i, 0)),
        out_shape=jax.ShapeDtypeStruct((n2, h), jnp.float32),
    )(parts, parts, ys, dinv, W, b)


def _tc_final(parts, ys3, dinv, W3, b, n_real):
    n2, d = ys3.shape
    d3 = W3.shape[1]

    def body(r0_ref, r1_ref, ys_ref, dv_ref, w3_ref, b_ref, h_ref, xg_ref):
        t = r0_ref[0] + r1_ref[0] + ys_ref[...]
        t = jnp.dot(t, w3_ref[...], preferred_element_type=jnp.float32)
        hb = dv_ref[...] * t + b_ref[...]
        h_ref[...] = hb

        @pl.when(pl.program_id(0) == 0)
        def _():
            xg_ref[...] = jnp.zeros_like(xg_ref)

        row = (pl.program_id(0) * _BN
               + lax.broadcasted_iota(jnp.int32, (_BN, 1), 0))
        hm = jnp.where(row < n_real, hb, 0.0)
        xg_ref[...] += jnp.sum(hm, axis=0, keepdims=True) * (1.0 / n_real)

    return pl.pallas_call(
        body,
        grid=(n2 // _BN,),
        in_specs=[
            pl.BlockSpec((1, _BN, d), lambda i: (0, i, 0)),
            pl.BlockSpec((1, _BN, d), lambda i: (1, i, 0)),
            pl.BlockSpec((_BN, d), lambda i: (i, 0)),
            pl.BlockSpec((_BN, 1), lambda i: (i, 0)),
            pl.BlockSpec((d, d3), lambda i: (0, 0)),
            pl.BlockSpec((1, d3), lambda i: (0, 0)),
        ],
        out_specs=[
            pl.BlockSpec((_BN, d3), lambda i: (i, 0)),
            pl.BlockSpec((1, d3), lambda i: (0, 0)),
        ],
        out_shape=[
            jax.ShapeDtypeStruct((n2, d3), jnp.float32),
            jax.ShapeDtypeStruct((1, d3), jnp.float32),
        ],
    )(parts, parts, ys3, dinv, W3, b)


def kernel(x, edge_index, W1, b1, W2, b2, W3, b3):
    n, d_in = x.shape
    e = edge_index.shape[1]
    nchunks = e // _CHUNK

    n2 = -(-n // _BN) * _BN
    x_p = jnp.concatenate(
        [x, jnp.zeros((n2 - n, d_in), x.dtype)], axis=0)

    npc = -(-nchunks // (_NW * _BATCH)) * (_NW * _BATCH)
    pad = npc * _CHUNK - e
    pad_ar = jnp.arange(pad, dtype=edge_index.dtype)
    src_p = jnp.concatenate([edge_index[0], n + pad_ar % (n2 - n)])
    dst_p = jnp.concatenate([edge_index[1], pad_ar % n2])
    src2 = src_p.reshape(npc, _CHUNK)
    dst2p = dst_p.reshape(npc, _CHUNK)
    vals2 = jnp.concatenate(
        [jnp.ones((e,), jnp.float32),
         jnp.zeros((pad,), jnp.float32)]).reshape(npc, _CHUNK)

    deg_parts = _sc_degree(dst2p, vals2, n2).reshape(2, n2)
    p0 = deg_parts[0].reshape(n2, 1)
    p1 = deg_parts[1].reshape(n2, 1)
    dinv, xs = _tc_prep(p0, p1, x_p)

    s1 = _sc_agg(xs, src2, dst2p, n2)
    ys2 = _tc_layer(s1, xs, dinv, W1, b1.reshape(1, -1), n)

    s2 = _sc_agg(ys2, src2, dst2p, n2)
    ys3 = _tc_layer(s2, ys2, dinv, W2, b2.reshape(1, -1), n)

    s3 = _sc_agg(ys3, src2, dst2p, n2)
    h, xg = _tc_final(s3, ys3, dinv, W3, b3.reshape(1, -1), n)
    return (h[:n], xg)


# speedup vs baseline: 1.2411x; 1.0763x over previous
import functools

import jax
import jax.numpy as jnp
from jax import lax
from jax.experimental import pallas as pl
from jax.experimental.pallas import tpu as pltpu
from jax.experimental.pallas import tpu_sc as plsc

_CHUNK = 128
_NW = 32
_BATCH = 16
_BN = 2048


def _worker_id(cid, sid):
    return sid * 2 + cid


def _sc_degree(dst2p, vals2, n2):
    nchunks = dst2p.shape[0]
    per_w = nchunks // _NW
    nbat = per_w // _BATCH
    mesh = plsc.VectorSubcoreMesh(core_axis_name="c", subcore_axis_name="s")

    zc = 512

    @functools.partial(
        pl.kernel,
        out_type=jax.ShapeDtypeStruct((2 * n2,), jnp.float32),
        mesh=mesh,
        scratch_types=[
            pltpu.VMEM((_BATCH, _CHUNK), jnp.int32),
            pltpu.VMEM((_BATCH, _CHUNK), jnp.float32),
            pltpu.VMEM((zc,), jnp.float32),
            pltpu.VMEM_SHARED((n2,), jnp.float32),
            pltpu.SemaphoreType.DMA,
        ],
    )
    def k(dst_hbm, val_hbm, out_hbm, db, vb, buf, acc, sem):
        cid = lax.axis_index("c")
        sid = lax.axis_index("s")
        wid = _worker_id(cid, sid)

        def fill_zero(i, _):
            buf[pl.ds(i * 16, 16)] = jnp.zeros((16,), jnp.float32)
            return 0

        lax.fori_loop(0, zc // 16, fill_zero, 0)

        nzc = n2 // zc

        def zero_step(t, _):
            off = (sid + 16 * t) * zc
            pltpu.sync_copy(buf, acc.at[pl.ds(off, zc)])
            return 0

        lax.fori_loop(0, (nzc - sid + 15) // 16, zero_step, 0)
        plsc.subcore_barrier()

        base = wid * per_w

        def batch(t, _):
            cbase = base + t * _BATCH
            pltpu.sync_copy(dst_hbm.at[pl.ds(cbase, _BATCH)], db)
            pltpu.sync_copy(val_hbm.at[pl.ds(cbase, _BATCH)], vb)
            for j in range(_BATCH):
                pltpu.sync_copy(vb.at[j], acc.at[db.at[j]], add=True)
            return 0

        lax.fori_loop(0, nbat, batch, 0)
        plsc.subcore_barrier()

        def out_step(t, _):
            off = (sid + 16 * t) * zc
            pltpu.sync_copy(acc.at[pl.ds(off, zc)],
                            out_hbm.at[pl.ds(cid * n2 + off, zc)])
            return 0

        lax.fori_loop(0, (nzc - sid + 15) // 16, out_step, 0)

    return k(dst2p, vals2)


def _sc_agg(hs, src2, dst2, n2):
    nh, w = hs.shape
    nchunks = src2.shape[0]
    per_w = nchunks // _NW
    nbat = per_w // _BATCH
    mesh = plsc.VectorSubcoreMesh(core_axis_name="c", subcore_axis_name="s")

    zr = 40

    @functools.partial(
        pl.kernel,
        out_type=jax.ShapeDtypeStruct((2, n2, w), jnp.float32),
        mesh=mesh,
        scratch_types=[
            pltpu.VMEM((2, _BATCH, _CHUNK), jnp.int32),
            pltpu.VMEM((2, _BATCH, _CHUNK), jnp.int32),
            pltpu.VMEM((2, _CHUNK, w), jnp.float32),
            pltpu.VMEM((zr, w), jnp.float32),
            pltpu.VMEM_SHARED((n2, w), jnp.float32),
            pltpu.SemaphoreType.DMA,
            pltpu.SemaphoreType.DMA,
            pltpu.SemaphoreType.DMA,
        ],
    )
    def k(h_hbm, src_hbm, dst_hbm, out_hbm, sb, db, rows, buf, acc,
          sem0, sem1, isem):
        cid = lax.axis_index("c")
        sid = lax.axis_index("s")
        wid = _worker_id(cid, sid)
        sems = (sem0, sem1)

        def zrow(r, _):
            def zcol(c_, __):
                buf[r, pl.ds(c_ * 16, 16)] = jnp.zeros((16,), jnp.float32)
                return 0
            lax.fori_loop(0, w // 16, zcol, 0)
            return 0

        lax.fori_loop(0, zr, zrow, 0)

        nzc = n2 // zr
        nz = (nzc - sid + 15) // 16

        def zero_step(t, _):
            off = (sid + 16 * t) * zr
            pltpu.async_copy(buf, acc.at[pl.ds(off, zr)], sem1)
            return 0

        lax.fori_loop(0, nz, zero_step, 0)

        def zero_drain(t, _):
            pltpu.make_async_copy(buf, acc.at[pl.ds(sid * zr, zr)],
                                  sem1).wait()
            return 0

        lax.fori_loop(0, nz, zero_drain, 0)
        plsc.subcore_barrier()

        base = wid * per_w

        pltpu.sync_copy(src_hbm.at[pl.ds(base, _BATCH)], sb.at[0])
        pltpu.sync_copy(dst_hbm.at[pl.ds(base, _BATCH)], db.at[0])
        pltpu.async_copy(h_hbm.at[sb.at[0, 0]], rows.at[0], sem0)
        pltpu.async_copy(h_hbm.at[sb.at[0, 1]], rows.at[1], sem1)

        def batch(t, _):
            cur = t % 2
            nxt = 1 - cur

            @pl.when(t + 1 < nbat)
            def _():
                cb2 = base + (t + 1) * _BATCH
                pltpu.async_copy(src_hbm.at[pl.ds(cb2, _BATCH)],
                                 sb.at[nxt], isem)
                pltpu.async_copy(dst_hbm.at[pl.ds(cb2, _BATCH)],
                                 db.at[nxt], isem)

            for j in range(_BATCH):
                b = j & 1
                pltpu.make_async_copy(
                    h_hbm.at[sb.at[0, 0]], rows.at[b], sems[b]).wait()
                pltpu.sync_copy(rows.at[b], acc.at[db.at[cur, j]], add=True)
                jn = j + 2
                if jn < _BATCH:
                    pltpu.async_copy(
                        h_hbm.at[sb.at[cur, jn]], rows.at[b], sems[b])
                elif jn == _BATCH:
                    @pl.when(t + 1 < nbat)
                    def _():
                        pltpu.make_async_copy(
                            src_hbm.at[pl.ds(base, _BATCH)], sb.at[nxt],
                            isem).wait()
                        pltpu.make_async_copy(
                            dst_hbm.at[pl.ds(base, _BATCH)], db.at[nxt],
                            isem).wait()
                        pltpu.async_copy(
                            h_hbm.at[sb.at[nxt, 0]], rows.at[b], sems[b])
                else:
                    @pl.when(t + 1 < nbat)
                    def _():
                        pltpu.async_copy(
                            h_hbm.at[sb.at[nxt, 1]], rows.at[b], sems[b])
            return 0

        lax.fori_loop(0, nbat, batch, 0)
        plsc.subcore_barrier()

        def out_step(t, _):
            off = (sid + 16 * t) * zr
            pltpu.async_copy(acc.at[pl.ds(off, zr)],
                             out_hbm.at[cid, pl.ds(off, zr)], sem1)
            return 0

        lax.fori_loop(0, nz, out_step, 0)

        def out_drain(t, _):
            pltpu.make_async_copy(acc.at[pl.ds(sid * zr, zr)],
                                  out_hbm.at[cid, pl.ds(sid * zr, zr)],
                                  sem1).wait()
            return 0

        lax.fori_loop(0, nz, out_drain, 0)

    return k(hs, src2, dst2)


def _tc_prep(p0, p1, x):
    n2, d = x.shape

    def body(p0_ref, p1_ref, x_ref, dinv_ref, xs_ref):
        deg = p0_ref[...] + p1_ref[...] + 1.0
        dv = lax.rsqrt(deg)
        dinv_ref[...] = dv
        xs_ref[...] = x_ref[...] * dv

    return pl.pallas_call(
        body,
        grid=(n2 // _BN,),
        in_specs=[
            pl.BlockSpec((_BN, 1), lambda i: (i, 0)),
            pl.BlockSpec((_BN, 1), lambda i: (i, 0)),
            pl.BlockSpec((_BN, d), lambda i: (i, 0)),
        ],
        out_specs=[
            pl.BlockSpec((_BN, 1), lambda i: (i, 0)),
            pl.BlockSpec((_BN, d), lambda i: (i, 0)),
        ],
        out_shape=[
            jax.ShapeDtypeStruct((n2, 1), jnp.float32),
            jax.ShapeDtypeStruct((n2, d), jnp.float32),
        ],
    )(p0, p1, x)


def _tc_layer(parts, ys, dinv, W, b, n_real):
    n2, d = ys.shape
    h = W.shape[1]

    def body(q0_ref, q1_ref, ys_ref, dv_ref, w_ref, b_ref, out_ref):
        t = q0_ref[0] + q1_ref[0] + ys_ref[...]
        t = jnp.dot(t, w_ref[...], preferred_element_type=jnp.float32)
        dv = dv_ref[...]
        out = jnp.maximum(dv * t + b_ref[...], 0.0)
        row = (pl.program_id(0) * _BN
               + lax.broadcasted_iota(jnp.int32, (_BN, 1), 0))
        out_ref[...] = jnp.where(row < n_real, dv * out, 0.0)

    return pl.pallas_call(
        body,
        grid=(n2 // _BN,),
        in_specs=[
            pl.BlockSpec((1, _BN, d), lambda i: (0, i, 0)),
            pl.BlockSpec((1, _BN, d), lambda i: (1, i, 0)),
            pl.BlockSpec((_BN, d), lambda i: (i, 0)),
            pl.BlockSpec((_BN, 1), lambda i: (i, 0)),
            pl.BlockSpec((d, h), lambda i: (0, 0)),
            pl.BlockSpec((1, h), lambda i: (0, 0)),
        ],
        out_specs=pl.BlockSpec((_BN, h), lambda i: (i, 0)),
        out_shape=jax.ShapeDtypeStruct((n2, h), jnp.float32),
    )(parts, parts, ys, dinv, W, b)


def _tc_final(parts, ys3, dinv, W3, b, n_real):
    n2, d = ys3.shape
    d3 = W3.shape[1]

    def body(r0_ref, r1_ref, ys_ref, dv_ref, w3_ref, b_ref, h_ref, xg_ref):
        t = r0_ref[0] + r1_ref[0] + ys_ref[...]
        t = jnp.dot(t, w3_ref[...], preferred_element_type=jnp.float32)
        hb = dv_ref[...] * t + b_ref[...]
        h_ref[...] = hb

        @pl.when(pl.program_id(0) == 0)
        def _():
            xg_ref[...] = jnp.zeros_like(xg_ref)

        row = (pl.program_id(0) * _BN
               + lax.broadcasted_iota(jnp.int32, (_BN, 1), 0))
        hm = jnp.where(row < n_real, hb, 0.0)
        xg_ref[...] += jnp.sum(hm, axis=0, keepdims=True) * (1.0 / n_real)

    return pl.pallas_call(
        body,
        grid=(n2 // _BN,),
        in_specs=[
            pl.BlockSpec((1, _BN, d), lambda i: (0, i, 0)),
            pl.BlockSpec((1, _BN, d), lambda i: (1, i, 0)),
            pl.BlockSpec((_BN, d), lambda i: (i, 0)),
            pl.BlockSpec((_BN, 1), lambda i: (i, 0)),
            pl.BlockSpec((d, d3), lambda i: (0, 0)),
            pl.BlockSpec((1, d3), lambda i: (0, 0)),
        ],
        out_specs=[
            pl.BlockSpec((_BN, d3), lambda i: (i, 0)),
            pl.BlockSpec((1, d3), lambda i: (0, 0)),
        ],
        out_shape=[
            jax.ShapeDtypeStruct((n2, d3), jnp.float32),
            jax.ShapeDtypeStruct((1, d3), jnp.float32),
        ],
    )(parts, parts, ys3, dinv, W3, b)


def kernel(x, edge_index, W1, b1, W2, b2, W3, b3):
    n, d_in = x.shape
    e = edge_index.shape[1]
    nchunks = e // _CHUNK

    n2 = -(-n // _BN) * _BN
    x_p = jnp.concatenate(
        [x, jnp.zeros((n2 - n, d_in), x.dtype)], axis=0)

    npc = -(-nchunks // (_NW * _BATCH)) * (_NW * _BATCH)
    pad = npc * _CHUNK - e
    pad_ar = jnp.arange(pad, dtype=edge_index.dtype)
    src_p = jnp.concatenate([edge_index[0], n + pad_ar % (n2 - n)])
    dst_p = jnp.concatenate([edge_index[1], pad_ar % n2])
    src2 = src_p.reshape(npc, _CHUNK)
    dst2p = dst_p.reshape(npc, _CHUNK)
    vals2 = jnp.concatenate(
        [jnp.ones((e,), jnp.float32),
         jnp.zeros((pad,), jnp.float32)]).reshape(npc, _CHUNK)

    deg_parts = _sc_degree(dst2p, vals2, n2).reshape(2, n2)
    p0 = deg_parts[0].reshape(n2, 1)
    p1 = deg_parts[1].reshape(n2, 1)
    dinv, xs = _tc_prep(p0, p1, x_p)

    s1 = _sc_agg(xs, src2, dst2p, n2)
    ys2 = _tc_layer(s1, xs, dinv, W1, b1.reshape(1, -1), n)

    s2 = _sc_agg(ys2, src2, dst2p, n2)
    ys3 = _tc_layer(s2, ys2, dinv, W2, b2.reshape(1, -1), n)

    s3 = _sc_agg(ys3, src2, dst2p, n2)
    h, xg = _tc_final(s3, ys3, dinv, W3, b3.reshape(1, -1), n)
    return (h[:n], xg)
